# baseline (device time: 7271 ns/iter reference)
import jax
import jax.numpy as jnp
from jax import lax
from jax.experimental import pallas as pl
from jax.experimental.pallas import tpu as pltpu

N_DEV = 4
N_CHUNK = 2


def kernel(x):
    m_rows, n_per = x.shape
    rows_c = m_rows // N_CHUNK

    def body(x_ref, out_ref, stats_ref, send_sems, recv_sems):
        my = lax.axis_index("i")

        barrier_sem = pltpu.get_barrier_semaphore()
        for d in range(1, N_DEV):
            pl.semaphore_signal(
                barrier_sem,
                inc=1,
                device_id=((my + d) % N_DEV,),
                device_id_type=pl.DeviceIdType.MESH,
            )

        es = []
        lmaxs = []
        rdmas = []
        for c in range(N_CHUNK):
            rs = pl.ds(c * rows_c, rows_c)
            xf = x_ref[rs, :].astype(jnp.float32)
            lmax = jnp.max(xf, axis=1)
            e = jnp.exp(xf - lmax[:, None])
            lsum = jnp.sum(e, axis=1)
            es.append(e)
            lmaxs.append(lmax)
            stats_ref[N_DEV - 1, :, rs] = jnp.stack([lmax, lsum])
            if c == 0:
                pl.semaphore_wait(barrier_sem, N_DEV - 1)
            cr = []
            for d in range(1, N_DEV):
                rdma = pltpu.make_async_remote_copy(
                    src_ref=stats_ref.at[N_DEV - 1, :, rs],
                    dst_ref=stats_ref.at[d - 1, :, rs],
                    send_sem=send_sems.at[c, d - 1],
                    recv_sem=recv_sems.at[c, d - 1],
                    device_id=((my + d) % N_DEV,),
                    device_id_type=pl.DeviceIdType.MESH,
                )
                rdma.start()
                cr.append(rdma)
            rdmas.append(cr)

        for c in range(N_CHUNK):
            rs = pl.ds(c * rows_c, rows_c)
            for rdma in rdmas[c]:
                rdma.wait()
            stats = stats_ref[:, :, rs]
            ms = stats[:, 0, :]
            ss = stats[:, 1, :]
            gmax = jnp.max(ms, axis=0)
            gsum = jnp.sum(ss * jnp.exp(ms - gmax[None, :]), axis=0)
            scale = jnp.exp(lmaxs[c] - gmax) / gsum
            out_ref[rs, :] = (es[c] * scale[:, None]).astype(out_ref.dtype)


    return pl.pallas_call(
        body,
        out_shape=jax.ShapeDtypeStruct((m_rows, n_per), jnp.float32),
        in_specs=[pl.BlockSpec(memory_space=pltpu.VMEM)],
        out_specs=pl.BlockSpec(memory_space=pltpu.VMEM),
        scratch_shapes=[
            pltpu.VMEM((N_DEV, 2, m_rows), jnp.float32),
            pltpu.SemaphoreType.DMA((N_CHUNK, N_DEV - 1)),
            pltpu.SemaphoreType.DMA((N_CHUNK, N_DEV - 1)),
        ],
        compiler_params=pltpu.CompilerParams(collective_id=0),
    )(x)


# device time: 7198 ns/iter; 1.0101x vs baseline; 1.0101x over previous
import jax
import jax.numpy as jnp
from jax import lax
from jax.experimental import pallas as pl
from jax.experimental.pallas import tpu as pltpu

N_DEV = 4


def kernel(x):
    m_rows, n_per = x.shape

    def body(
        x_hbm,
        out_hbm,
        xbuf,
        obuf,
        stats_ref,
        send_sems,
        recv_sems,
        in_sem,
        out_sem,
    ):
        my = lax.axis_index("i")

        cp_in = pltpu.make_async_copy(x_hbm, xbuf, in_sem)
        cp_in.start()

        barrier_sem = pltpu.get_barrier_semaphore()
        for d in range(1, N_DEV):
            pl.semaphore_signal(
                barrier_sem,
                inc=1,
                device_id=((my + d) % N_DEV,),
                device_id_type=pl.DeviceIdType.MESH,
            )

        cp_in.wait()

        xf = xbuf[:, :].astype(jnp.float32)
        lmax = jnp.max(xf, axis=1)
        e = jnp.exp(xf - lmax[:, None])
        lsum = jnp.sum(e, axis=1)

        stats_ref[N_DEV - 1, :, :] = jnp.stack([lmax, lsum])

        pl.semaphore_wait(barrier_sem, N_DEV - 1)

        rdmas = []
        for d in range(1, N_DEV):
            rdma = pltpu.make_async_remote_copy(
                src_ref=stats_ref.at[N_DEV - 1],
                dst_ref=stats_ref.at[d - 1],
                send_sem=send_sems.at[d - 1],
                recv_sem=recv_sems.at[d - 1],
                device_id=((my + d) % N_DEV,),
                device_id_type=pl.DeviceIdType.MESH,
            )
            rdma.start()
            rdmas.append(rdma)
        for rdma in rdmas:
            rdma.wait()

        stats = stats_ref[:, :, :]
        ms = stats[:, 0, :]
        ss = stats[:, 1, :]
        gmax = jnp.max(ms, axis=0)
        gsum = jnp.sum(ss * jnp.exp(ms - gmax[None, :]), axis=0)

        scale = jnp.exp(lmax - gmax) / gsum
        obuf[:, :] = (e * scale[:, None]).astype(obuf.dtype)

        cp_out = pltpu.make_async_copy(obuf, out_hbm, out_sem)
        cp_out.start()
        cp_out.wait()


    return pl.pallas_call(
        body,
        out_shape=jax.ShapeDtypeStruct((m_rows, n_per), jnp.float32),
        in_specs=[pl.BlockSpec(memory_space=pl.ANY)],
        out_specs=pl.BlockSpec(memory_space=pl.ANY),
        scratch_shapes=[
            pltpu.VMEM((m_rows, n_per), jnp.float32),
            pltpu.VMEM((m_rows, n_per), jnp.float32),
            pltpu.VMEM((N_DEV, 2, m_rows), jnp.float32),
            pltpu.SemaphoreType.DMA((N_DEV - 1,)),
            pltpu.SemaphoreType.DMA((N_DEV - 1,)),
            pltpu.SemaphoreType.DMA,
            pltpu.SemaphoreType.DMA,
        ],
        compiler_params=pltpu.CompilerParams(collective_id=0),
    )(x)
